# baseline (speedup 1.0000x reference)
import functools

import jax
import jax.numpy as jnp
from jax import lax
from jax.experimental import pallas as pl
from jax.experimental.pallas import tpu as pltpu
from jax.experimental.pallas import tpu_sc as plsc

_NA, _NC = 5, 20
_NX, _NY = 32, 32
_S = _NX * _NY
_T = 128
_BS = 16
_CH = _NA * (5 + _NC)
_CPA = 5 + _NC

_IGNORE = 0.5
_OBJ_SCALE = 5.0

_IPS = 4
_NCORES, _NSUB, _LANES = 2, 16, 16

_INTERPRET = False


def _sc_table(xflat, ttile, anchors):
    mesh = plsc.VectorSubcoreMesh(
        core_axis_name="c", subcore_axis_name="s",
        num_cores=_NCORES, num_subcores=_NSUB)

    @functools.partial(
        pl.kernel,
        out_type=jax.ShapeDtypeStruct((_T // _LANES, 32, _LANES), jnp.float32),
        mesh=mesh,
        scratch_types=[
            pltpu.VMEM((6, _LANES), jnp.float32),
            pltpu.VMEM((2 * _NA, _LANES), jnp.float32),
            pltpu.VMEM((32, _LANES), jnp.float32),
            pltpu.SemaphoreType.DMA,
        ],
    )
    def k(xflat_hbm, ttile_hbm, anc_hbm, out_hbm, tbuf, abuf, obuf, gsem):
        wid = lax.axis_index("s") * _NCORES + lax.axis_index("c")

        @pl.when(wid < _T // _LANES)
        def _():
            pltpu.sync_copy(ttile_hbm.at[wid], tbuf)
            pltpu.sync_copy(anc_hbm, abuf)
            img = tbuf[0, :]
            gx = tbuf[2, :] * float(_NX)
            gy = tbuf[3, :] * float(_NY)
            gw = tbuf[4, :] * float(_NX)
            gh = tbuf[5, :] * float(_NY)
            gx1, gx2 = gx - 0.5 * gw, gx + 0.5 * gw
            gy1, gy2 = gy - 0.5 * gh, gy + 0.5 * gh
            garea = gw * gh
            cxi = jnp.minimum(jnp.maximum(gx.astype(jnp.int32), 0), _NX - 1)
            cyi = jnp.minimum(jnp.maximum(gy.astype(jnp.int32), 0), _NY - 1)
            cellxf = cxi.astype(jnp.float32)
            cellyf = cyi.astype(jnp.float32)
            cell = cyi * _NX + cxi

            best = jnp.zeros((_LANES,), jnp.int32)
            bestv = jnp.full((_LANES,), -1.0, jnp.float32)
            baw = jnp.zeros((_LANES,), jnp.float32)
            bah = jnp.zeros((_LANES,), jnp.float32)
            for a in range(_NA):
                aw = abuf[2 * a, :]
                ah = abuf[2 * a + 1, :]
                iw = jnp.maximum(
                    jnp.minimum(cellxf + 0.5 * aw, gx2)
                    - jnp.maximum(cellxf - 0.5 * aw, gx1), 0.0)
                ih = jnp.maximum(
                    jnp.minimum(cellyf + 0.5 * ah, gy2)
                    - jnp.maximum(cellyf - 0.5 * ah, gy1), 0.0)
                inter = iw * ih
                iou = inter / (aw * ah + garea - inter + 1e-9)
                upd = iou > bestv
                best = jnp.where(upd, a, best)
                bestv = jnp.where(upd, iou, bestv)
                baw = jnp.where(upd, aw, baw)
                bah = jnp.where(upd, ah, bah)

            idx0 = (img.astype(jnp.int32) * (_CH * _S)
                    + best * (_CPA * _S) + cell)
            handles = []
            for kk in range(_CPA):
                handles.append(pltpu.async_copy(
                    xflat_hbm.at[idx0 + kk * _S], obuf.at[kk], gsem))
            for h in handles:
                h.wait()
            obuf[25, :] = baw
            obuf[26, :] = bah
            obuf[27, :] = cellxf
            obuf[28, :] = cellyf
            obuf[29, :] = best.astype(jnp.float32)
            obuf[30, :] = jnp.zeros((_LANES,), jnp.float32)
            obuf[31, :] = jnp.zeros((_LANES,), jnp.float32)
            pltpu.sync_copy(obuf, out_hbm.at[wid])

    return k(xflat, ttile, anchors)


def _dense_kernel(x_ref, tcol_ref, trow_ref, anc_ref, out_ref, sd_ref):
    b = pl.program_id(0)

    img_c = tcol_ref[:, 0:1]
    iota_c = jax.lax.broadcasted_iota(jnp.int32, (_T, 1), 0)

    @pl.when(b == 0)
    def _init():
        out_ref[...] = jnp.zeros_like(out_ref)
        gx_c = tcol_ref[:, 2:3] * _NX
        gy_c = tcol_ref[:, 3:4] * _NY
        gw_c = tcol_ref[:, 4:5] * _NX
        gh_c = tcol_ref[:, 5:6] * _NY
        gx1_c, gx2_c = gx_c - 0.5 * gw_c, gx_c + 0.5 * gw_c
        gy1_c, gy2_c = gy_c - 0.5 * gh_c, gy_c + 0.5 * gh_c
        garea_c = gw_c * gh_c
        img_r = trow_ref[0:1, :]
        jt = jax.lax.broadcasted_iota(jnp.int32, (1, _T), 1)
        before = (img_c < img_r) | ((img_c == img_r) & (iota_c < jt))
        rank_r = jnp.sum(before.astype(jnp.float32), axis=0, keepdims=True)
        perm = (rank_r == iota_c.astype(jnp.float32)).astype(jnp.float32)
        zeros3 = jnp.zeros((_T, 3), jnp.float32)
        d = jnp.concatenate(
            [gx1_c, gy1_c, gx2_c, gy2_c, garea_c + 1e-9, zeros3],
            axis=1)
        sd_ref[...] = jax.lax.dot_general(
            perm, d, (((1,), (0,)), ((), ())),
            precision=jax.lax.Precision.HIGHEST,
            preferred_element_type=jnp.float32)

    lane = jax.lax.broadcasted_iota(jnp.int32, (1, _S), 1)
    sxf = (lane % _NX).astype(jnp.float32)
    syf = (lane // _NX).astype(jnp.float32)
    iota8 = jax.lax.broadcasted_iota(jnp.int32, (8, 1), 0)

    nadd = jnp.zeros((1, _S), jnp.float32)
    for i in range(_IPS):
        bi = b * _IPS + i
        bif = bi.astype(jnp.float32)
        xb = x_ref[i]

        cnt = jnp.sum((img_c == bif).astype(jnp.int32))
        start = jnp.sum((img_c < bif).astype(jnp.int32))
        end = start + cnt
        nchunks = (cnt + 7) // 8

        px1s, px2s, py1s, py2s, pareas, confs = [], [], [], [], [], []
        for a in range(_NA):
            aw = anc_ref[a:a + 1, 0:1]
            ah = anc_ref[a:a + 1, 1:2]
            base = a * _CPA
            cx = jax.nn.sigmoid(xb[base + 0:base + 1, :]) + sxf
            cy = jax.nn.sigmoid(xb[base + 1:base + 2, :]) + syf
            w = jnp.exp(jnp.clip(xb[base + 2:base + 3, :], -10.0, 10.0)) * aw
            h = jnp.exp(jnp.clip(xb[base + 3:base + 4, :], -10.0, 10.0)) * ah
            confs.append(jax.nn.sigmoid(xb[base + 4:base + 5, :]))
            px1s.append(cx - 0.5 * w)
            px2s.append(cx + 0.5 * w)
            py1s.append(cy - 0.5 * h)
            py2s.append(cy + 0.5 * h)
            pareas.append(w * h)

        def chunk_body(c, carry, start=start, end=end, px1s=px1s, px2s=px2s,
                       py1s=py1s, py2s=py2s, pareas=pareas):
            nominal = start + 8 * c
            off = jnp.minimum(nominal, _T - 8)
            rows = sd_ref[pl.ds(off, 8), :]
            sgx1, sgy1 = rows[:, 0:1], rows[:, 1:2]
            sgx2, sgy2 = rows[:, 2:3], rows[:, 3:4]
            sgoff = rows[:, 4:5]
            ridx = iota8 + off
            inb = (ridx >= nominal) & (ridx < end)
            den = sgoff + jnp.where(inb, 0.0, 1e9)
            outs = []
            for a in range(_NA):
                wi = jnp.maximum(
                    jnp.minimum(px2s[a], sgx2) - jnp.maximum(px1s[a], sgx1), 0.0)
                hi = jnp.maximum(
                    jnp.minimum(py2s[a], sgy2) - jnp.maximum(py1s[a], sgy1), 0.0)
                score = 3.0 * (wi * hi) - (pareas[a] + den)
                outs.append(jnp.maximum(carry[a], score))
            return tuple(outs)

        init = tuple(jnp.full((8, _S), -1.0, jnp.float32) for _ in range(_NA))
        smaxs = jax.lax.fori_loop(0, nchunks, chunk_body, init)
        for a in range(_NA):
            smax = jnp.max(smaxs[a], axis=0, keepdims=True)
            nadd += jnp.where(smax < 0.0, confs[a] * confs[a], 0.0)
    out_ref[...] += jnp.broadcast_to(nadd, (8, _S))


def _combine_kernel(tab_ref, nacc_ref, tcol_ref, trow_ref, anc_ref, out_ref):
    iota_c = jax.lax.broadcasted_iota(jnp.int32, (_T, 1), 0)
    jt = jax.lax.broadcasted_iota(jnp.int32, (1, _T), 1)

    img_r = trow_ref[0:1, :]
    cls_r = trow_ref[1:2, :].astype(jnp.int32)
    gx_r = trow_ref[2:3, :] * _NX
    gy_r = trow_ref[3:4, :] * _NY
    gw_r = trow_ref[4:5, :] * _NX
    gh_r = trow_ref[5:6, :] * _NY
    gx1_r, gx2_r = gx_r - 0.5 * gw_r, gx_r + 0.5 * gw_r
    gy1_r, gy2_r = gy_r - 0.5 * gh_r, gy_r + 0.5 * gh_r
    garea_r = gw_r * gh_r
    cellx_r = jnp.clip(jnp.floor(gx_r), 0.0, _NX - 1.0)
    celly_r = jnp.clip(jnp.floor(gy_r), 0.0, _NY - 1.0)
    cell_r = (celly_r * _NX + cellx_r).astype(jnp.int32)

    best_r = jnp.zeros((1, _T), jnp.int32)
    bestv_r = jnp.full((1, _T), -1.0, jnp.float32)
    baw_r = jnp.zeros((1, _T), jnp.float32)
    bah_r = jnp.zeros((1, _T), jnp.float32)
    for a in range(_NA):
        aw = anc_ref[a:a + 1, 0:1]
        ah = anc_ref[a:a + 1, 1:2]
        iw = jnp.maximum(
            jnp.minimum(cellx_r + 0.5 * aw, gx2_r)
            - jnp.maximum(cellx_r - 0.5 * aw, gx1_r), 0.0)
        ih = jnp.maximum(
            jnp.minimum(celly_r + 0.5 * ah, gy2_r)
            - jnp.maximum(celly_r - 0.5 * ah, gy1_r), 0.0)
        inter = iw * ih
        iou = inter / (aw * ah + garea_r - inter + 1e-9)
        upd = iou > bestv_r
        best_r = jnp.where(upd, a, best_r)
        bestv_r = jnp.where(upd, iou, bestv_r)
        baw_r = jnp.where(upd, jnp.broadcast_to(aw, (1, _T)), baw_r)
        bah_r = jnp.where(upd, jnp.broadcast_to(ah, (1, _T)), bah_r)

    g32 = jnp.concatenate([tab_ref[w] for w in range(_T // _LANES)],
                          axis=1)
    g = g32[0:_CPA, :]
    sx_g = jax.nn.sigmoid(g[0:1, :])
    sy_g = jax.nn.sigmoid(g[1:2, :])
    ew_g = jnp.exp(jnp.clip(g[2:3, :], -10.0, 10.0))
    eh_g = jnp.exp(jnp.clip(g[3:4, :], -10.0, 10.0))
    conf_g = jax.nn.sigmoid(g[4:5, :])
    logits = g[5:5 + _NC, :]

    img_c = tcol_ref[:, 0:1]
    gx_c = tcol_ref[:, 2:3] * _NX
    gy_c = tcol_ref[:, 3:4] * _NY
    gw_c = tcol_ref[:, 4:5] * _NX
    gh_c = tcol_ref[:, 5:6] * _NY
    gx1_c, gx2_c = gx_c - 0.5 * gw_c, gx_c + 0.5 * gw_c
    gy1_c, gy2_c = gy_c - 0.5 * gh_c, gy_c + 0.5 * gh_c
    garea_c = gw_c * gh_c
    cellx_c = jnp.clip(jnp.floor(gx_c), 0.0, _NX - 1.0)
    celly_c = jnp.clip(jnp.floor(gy_c), 0.0, _NY - 1.0)
    cell_c = (celly_c * _NX + cellx_c).astype(jnp.int32)

    w_p = ew_g * baw_r
    h_p = eh_g * bah_r
    cx_p = sx_g + cellx_r
    cy_p = sy_g + celly_r
    px1_p, px2_p = cx_p - 0.5 * w_p, cx_p + 0.5 * w_p
    py1_p, py2_p = cy_p - 0.5 * h_p, cy_p + 0.5 * h_p
    wi_p = jnp.maximum(jnp.minimum(px2_p, gx2_c) - jnp.maximum(px1_p, gx1_c), 0.0)
    hi_p = jnp.maximum(jnp.minimum(py2_p, gy2_c) - jnp.maximum(py1_p, gy1_c), 0.0)
    inter_p = wi_p * hi_p
    iou_p = inter_p / (w_p * h_p + garea_c - inter_p + 1e-9)
    iou_p = jnp.where(img_c == img_r, iou_p, 0.0)
    miou_g = jnp.max(iou_p, axis=0, keepdims=True)

    best_c = jnp.zeros((_T, 1), jnp.int32)
    bestv_c = jnp.full((_T, 1), -1.0, jnp.float32)
    for a in range(_NA):
        aw = anc_ref[a:a + 1, 0:1]
        ah = anc_ref[a:a + 1, 1:2]
        iw = jnp.maximum(
            jnp.minimum(cellx_c + 0.5 * aw, gx2_c)
            - jnp.maximum(cellx_c - 0.5 * aw, gx1_c), 0.0)
        ih = jnp.maximum(
            jnp.minimum(celly_c + 0.5 * ah, gy2_c)
            - jnp.maximum(celly_c - 0.5 * ah, gy1_c), 0.0)
        inter = iw * ih
        iou = inter / (aw * ah + garea_c - inter + 1e-9)
        upd = iou > bestv_c
        best_c = jnp.where(upd, a, best_c)
        bestv_c = jnp.where(upd, iou, bestv_c)
    key_c = img_c.astype(jnp.int32) * (_NA * _S) + best_c * _S + cell_c
    key_r = img_r.astype(jnp.int32) * (_NA * _S) + best_r * _S + cell_r
    dup = ((key_c == key_r) & (iota_c > jt)).astype(jnp.float32)
    later = jnp.max(dup, axis=0, keepdims=True)
    valid = jnp.where(later > 0.0, 0.0, 1.0)

    tx = gx_r - cellx_r
    ty = gy_r - celly_r
    tw = gw_r / baw_r
    th = gh_r / bah_r
    pw = w_p * (1.0 / _NX)
    ph = h_p * (1.0 / _NY)
    bscale = 2.0 - pw * ph
    box = bscale * ((sx_g - tx) ** 2 + (sy_g - ty) ** 2
                    + (ew_g - tw) ** 2 + (eh_g - th) ** 2)
    obj = _OBJ_SCALE * (conf_g - miou_g) ** 2

    m = jnp.max(logits, axis=0, keepdims=True)
    lse = m + jnp.log(jnp.sum(jnp.exp(logits - m), axis=0, keepdims=True))
    ci = jax.lax.broadcasted_iota(jnp.int32, (_NC, _T), 0)
    sel = jnp.sum(jnp.where(ci == cls_r, logits, 0.0), axis=0, keepdims=True)
    ce = lse - sel

    nocorr = jnp.where(miou_g < _IGNORE, conf_g * conf_g, 0.0)
    possum = jnp.sum(valid * (box + obj + ce - nocorr),
                     axis=(0, 1), keepdims=True)
    nsum = jnp.sum(nacc_ref[...], axis=(0, 1), keepdims=True) * (1.0 / 8.0)
    out_ref[...] = (nsum + possum) / _BS


def kernel(p, targets, anchors):
    x = p.reshape(_BS, _CH, _S)
    xflat = p.reshape(_BS * _CH * _S)
    tcol = targets
    trow = targets.T

    ancx = jnp.broadcast_to(anchors.reshape(2 * _NA, 1), (2 * _NA, _LANES))
    ttile = trow.reshape(6, _T // _LANES, _LANES).transpose(1, 0, 2)
    table = _sc_table(xflat, ttile, ancx)

    nacc = pl.pallas_call(
        _dense_kernel,
        grid=(_BS // _IPS,),
        in_specs=[
            pl.BlockSpec((_IPS, _CH, _S), lambda b: (b, 0, 0)),
            pl.BlockSpec((_T, 6), lambda b: (0, 0)),
            pl.BlockSpec((6, _T), lambda b: (0, 0)),
            pl.BlockSpec((_NA, 2), lambda b: (0, 0)),
        ],
        out_specs=pl.BlockSpec((8, _S), lambda b: (0, 0)),
        out_shape=jax.ShapeDtypeStruct((8, _S), jnp.float32),
        scratch_shapes=[pltpu.VMEM((_T, 8), jnp.float32)],
        compiler_params=pltpu.CompilerParams(
            dimension_semantics=("arbitrary",)),
        interpret=_INTERPRET,
    )(x, tcol, trow, anchors)

    out = pl.pallas_call(
        _combine_kernel,
        in_specs=[
            pl.BlockSpec((_T // _LANES, 32, _LANES), lambda: (0, 0, 0)),
            pl.BlockSpec((8, _S), lambda: (0, 0)),
            pl.BlockSpec((_T, 6), lambda: (0, 0)),
            pl.BlockSpec((6, _T), lambda: (0, 0)),
            pl.BlockSpec((_NA, 2), lambda: (0, 0)),
        ],
        out_specs=pl.BlockSpec((1, 1), lambda: (0, 0)),
        out_shape=jax.ShapeDtypeStruct((1, 1), jnp.float32),
        interpret=_INTERPRET,
    )(table, nacc, tcol, trow, anchors)
    return out[0, 0]


# speedup vs baseline: 2.8827x; 2.8827x over previous
import jax
import jax.numpy as jnp
from jax.experimental import pallas as pl
from jax.experimental.pallas import tpu as pltpu

_NA, _NC = 5, 20
_NX, _NY = 32, 32
_S = _NX * _NY
_T = 128
_BS = 16
_CH = _NA * (5 + _NC)
_CPA = 5 + _NC

_IGNORE = 0.5
_OBJ_SCALE = 5.0

_IPS = 4
_INTERPRET = False


def _colprep(tcol_ref):
    img_c = tcol_ref[:, 0:1]
    gx_c = tcol_ref[:, 2:3] * _NX
    gy_c = tcol_ref[:, 3:4] * _NY
    gw_c = tcol_ref[:, 4:5] * _NX
    gh_c = tcol_ref[:, 5:6] * _NY
    gx1_c, gx2_c = gx_c - 0.5 * gw_c, gx_c + 0.5 * gw_c
    gy1_c, gy2_c = gy_c - 0.5 * gh_c, gy_c + 0.5 * gh_c
    garea_c = gw_c * gh_c
    cellx_c = jnp.clip(jnp.floor(gx_c), 0.0, _NX - 1.0)
    celly_c = jnp.clip(jnp.floor(gy_c), 0.0, _NY - 1.0)
    cell_c = (celly_c * _NX + cellx_c).astype(jnp.int32)
    return (img_c, gx_c, gy_c, gw_c, gh_c, gx1_c, gx2_c, gy1_c, gy2_c,
            garea_c, cellx_c, celly_c, cell_c)


def _loss_kernel(x_ref, tcol_ref, trow_ref, anc_ref, out_ref,
                 sd_ref, acc_ref, nacc_ref):
    b = pl.program_id(0)
    bf = b.astype(jnp.float32)

    img_c = tcol_ref[:, 0:1]
    iota_c = jax.lax.broadcasted_iota(jnp.int32, (_T, 1), 0)

    @pl.when(b == 0)
    def _init():
        acc_ref[...] = jnp.zeros_like(acc_ref)
        nacc_ref[...] = jnp.zeros_like(nacc_ref)
        (_, _, _, _, _, gx1_c, gx2_c, gy1_c, gy2_c,
         garea_c, _, _, cell_c) = _colprep(tcol_ref)
        img_r = trow_ref[0:1, :]
        jt = jax.lax.broadcasted_iota(jnp.int32, (1, _T), 1)
        before = (img_c < img_r) | ((img_c == img_r) & (iota_c < jt))
        rank_r = jnp.sum(before.astype(jnp.float32), axis=0, keepdims=True)
        perm = (rank_r == iota_c.astype(jnp.float32)).astype(jnp.float32)
        key2f = img_c * float(_S) + cell_c.astype(jnp.float32)
        zeros2 = jnp.zeros((_T, 2), jnp.float32)
        d = jnp.concatenate(
            [gx1_c, gy1_c, gx2_c, gy2_c, garea_c + 1e-9, key2f, zeros2],
            axis=1)
        sd_ref[...] = jax.lax.dot_general(
            perm, d, (((1,), (0,)), ((), ())),
            precision=jax.lax.Precision.HIGHEST,
            preferred_element_type=jnp.float32)

    lane = jax.lax.broadcasted_iota(jnp.int32, (1, _S), 1)
    sxf = (lane % _NX).astype(jnp.float32)
    syf = (lane // _NX).astype(jnp.float32)
    iota8 = jax.lax.broadcasted_iota(jnp.int32, (8, 1), 0)

    nadd = jnp.zeros((1, _S), jnp.float32)
    for i in range(_IPS):
        bi = b * _IPS + i
        bif = bi.astype(jnp.float32)
        xb = x_ref[i]
        featbf = xb.astype(jnp.bfloat16)

        cnt = jnp.sum((img_c == bif).astype(jnp.int32))
        start = jnp.sum((img_c < bif).astype(jnp.int32))
        end = start + cnt
        nchunks = (cnt + 7) // 8

        def stk(c):
            return jnp.concatenate(
                [xb[a * _CPA + c:a * _CPA + c + 1, :] for a in range(_NA)],
                axis=0)
        awc = anc_ref[:, 0:1]
        ahc = anc_ref[:, 1:2]
        cxs = jax.nn.sigmoid(stk(0)) + sxf
        cys = jax.nn.sigmoid(stk(1)) + syf
        ws = jnp.exp(jnp.clip(stk(2), -10.0, 10.0)) * awc
        hs = jnp.exp(jnp.clip(stk(3), -10.0, 10.0)) * ahc
        conf_s = jax.nn.sigmoid(stk(4))
        px1_s, px2_s = cxs - 0.5 * ws, cxs + 0.5 * ws
        py1_s, py2_s = cys - 0.5 * hs, cys + 0.5 * hs
        parea_s = ws * hs
        px1s = [px1_s[a:a + 1, :] for a in range(_NA)]
        px2s = [px2_s[a:a + 1, :] for a in range(_NA)]
        py1s = [py1_s[a:a + 1, :] for a in range(_NA)]
        py2s = [py2_s[a:a + 1, :] for a in range(_NA)]
        pareas = [parea_s[a:a + 1, :] for a in range(_NA)]

        keyrow = (lane + bi * _S).astype(jnp.float32)

        def chunk_body(c, carry, start=start, end=end, px1s=px1s, px2s=px2s,
                       py1s=py1s, py2s=py2s, pareas=pareas, keyrow=keyrow,
                       featbf=featbf):
            nominal = start + 8 * c
            off = jnp.minimum(nominal, _T - 8)
            rows = sd_ref[pl.ds(off, 8), :]
            sgx1, sgy1 = rows[:, 0:1], rows[:, 1:2]
            sgx2, sgy2 = rows[:, 2:3], rows[:, 3:4]
            sgoff, skey = rows[:, 4:5], rows[:, 5:6]
            ridx = iota8 + off
            inb = (ridx >= nominal) & (ridx < end)
            den = sgoff + jnp.where(inb, 0.0, 1e9)
            outs = []
            for a in range(_NA):
                wi = jnp.maximum(
                    jnp.minimum(px2s[a], sgx2) - jnp.maximum(px1s[a], sgx1), 0.0)
                hi = jnp.maximum(
                    jnp.minimum(py2s[a], sgy2) - jnp.maximum(py1s[a], sgy1), 0.0)
                score = 3.0 * (wi * hi) - (pareas[a] + den)
                outs.append(jnp.maximum(carry[a], score))
            onehot = ((skey == keyrow) & inb).astype(jnp.bfloat16)
            g8 = jax.lax.dot_general(
                onehot, featbf, (((1,), (1,)), ((), ())),
                preferred_element_type=jnp.float32)
            acc_ref[pl.ds(off, 8), :] = acc_ref[pl.ds(off, 8), :] + g8
            return tuple(outs)

        init = tuple(jnp.full((8, _S), -1.0, jnp.float32) for _ in range(_NA))
        smaxs = jax.lax.fori_loop(0, nchunks, chunk_body, init)
        smax_s = jnp.concatenate(
            [jnp.max(smaxs[a], axis=0, keepdims=True) for a in range(_NA)],
            axis=0)
        nadd += jnp.sum(jnp.where(smax_s < 0.0, conf_s * conf_s, 0.0),
                        axis=0, keepdims=True)
    nacc_ref[...] += jnp.broadcast_to(nadd, (8, _S))

    @pl.when(b == _BS // _IPS - 1)
    def _fin():
        (img_c2, gx_c, gy_c, gw_c, gh_c, gx1_c, gx2_c, gy1_c, gy2_c,
         garea_c, cellx_c, celly_c, cell_c) = _colprep(tcol_ref)
        cls_c = tcol_ref[:, 1:2].astype(jnp.int32)

        best_c = jnp.zeros((_T, 1), jnp.int32)
        bestv_c = jnp.full((_T, 1), -1.0, jnp.float32)
        baw_c = jnp.zeros((_T, 1), jnp.float32)
        bah_c = jnp.zeros((_T, 1), jnp.float32)
        for a in range(_NA):
            aw = anc_ref[a:a + 1, 0:1]
            ah = anc_ref[a:a + 1, 1:2]
            iw = jnp.maximum(
                jnp.minimum(cellx_c + 0.5 * aw, gx2_c)
                - jnp.maximum(cellx_c - 0.5 * aw, gx1_c), 0.0)
            ih = jnp.maximum(
                jnp.minimum(celly_c + 0.5 * ah, gy2_c)
                - jnp.maximum(celly_c - 0.5 * ah, gy1_c), 0.0)
            inter = iw * ih
            iou = inter / (aw * ah + garea_c - inter + 1e-9)
            upd = iou > bestv_c
            best_c = jnp.where(upd, a, best_c)
            bestv_c = jnp.where(upd, iou, bestv_c)
            baw_c = jnp.where(upd, jnp.broadcast_to(aw, (_T, 1)), baw_c)
            bah_c = jnp.where(upd, jnp.broadcast_to(ah, (_T, 1)), bah_c)

        img_r = trow_ref[0:1, :]
        jt = jax.lax.broadcasted_iota(jnp.int32, (1, _T), 1)
        before = (img_c2 < img_r) | ((img_c2 == img_r) & (iota_c < jt))
        rank_r = jnp.sum(before.astype(jnp.float32), axis=0, keepdims=True)
        perm = (rank_r == iota_c.astype(jnp.float32)).astype(jnp.float32)
        gall = jax.lax.dot_general(
            perm, acc_ref[...], (((0,), (0,)), ((), ())),
            precision=jax.lax.Precision.HIGHEST,
            preferred_element_type=jnp.float32)

        g25 = jnp.zeros((_T, _CPA), jnp.float32)
        for a in range(_NA):
            g25 = jnp.where(best_c == a, gall[:, a * _CPA:(a + 1) * _CPA], g25)

        sx_g = jax.nn.sigmoid(g25[:, 0:1])
        sy_g = jax.nn.sigmoid(g25[:, 1:2])
        ew_g = jnp.exp(jnp.clip(g25[:, 2:3], -10.0, 10.0))
        eh_g = jnp.exp(jnp.clip(g25[:, 3:4], -10.0, 10.0))
        conf_g = jax.nn.sigmoid(g25[:, 4:5])
        logits = g25[:, 5:5 + _NC]

        w_p = ew_g * baw_c
        h_p = eh_g * bah_c
        cx_p = sx_g + cellx_c
        cy_p = sy_g + celly_c
        px1_p, px2_p = cx_p - 0.5 * w_p, cx_p + 0.5 * w_p
        py1_p, py2_p = cy_p - 0.5 * h_p, cy_p + 0.5 * h_p
        gx1_r = trow_ref[2:3, :] * _NX - 0.5 * trow_ref[4:5, :] * _NX
        gx2_r = trow_ref[2:3, :] * _NX + 0.5 * trow_ref[4:5, :] * _NX
        gy1_r = trow_ref[3:4, :] * _NY - 0.5 * trow_ref[5:6, :] * _NY
        gy2_r = trow_ref[3:4, :] * _NY + 0.5 * trow_ref[5:6, :] * _NY
        garea_r = (trow_ref[4:5, :] * _NX) * (trow_ref[5:6, :] * _NY)
        wi_p = jnp.maximum(jnp.minimum(px2_p, gx2_r) - jnp.maximum(px1_p, gx1_r), 0.0)
        hi_p = jnp.maximum(jnp.minimum(py2_p, gy2_r) - jnp.maximum(py1_p, gy1_r), 0.0)
        inter_p = wi_p * hi_p
        iou_p = inter_p / (w_p * h_p + garea_r - inter_p + 1e-9)
        iou_p = jnp.where(img_c2 == img_r, iou_p, 0.0)
        miou_g = jnp.max(iou_p, axis=1, keepdims=True)

        key3_c = (img_c2.astype(jnp.int32) * (_NA * _S)
                  + best_c * _S + cell_c)
        cellx_rf = jnp.clip(jnp.floor(trow_ref[2:3, :] * _NX), 0.0, _NX - 1.0)
        celly_rf = jnp.clip(jnp.floor(trow_ref[3:4, :] * _NY), 0.0, _NY - 1.0)
        cell_rf = celly_rf * _NX + cellx_rf
        best_r = jnp.zeros((1, _T), jnp.int32)
        bestv_r = jnp.full((1, _T), -1.0, jnp.float32)
        for a in range(_NA):
            aw = anc_ref[a:a + 1, 0:1]
            ah = anc_ref[a:a + 1, 1:2]
            iw = jnp.maximum(
                jnp.minimum(cellx_rf + 0.5 * aw, gx2_r)
                - jnp.maximum(cellx_rf - 0.5 * aw, gx1_r), 0.0)
            ih = jnp.maximum(
                jnp.minimum(celly_rf + 0.5 * ah, gy2_r)
                - jnp.maximum(celly_rf - 0.5 * ah, gy1_r), 0.0)
            inter = iw * ih
            iou = inter / (aw * ah + garea_r - inter + 1e-9)
            upd = iou > bestv_r
            best_r = jnp.where(upd, a, best_r)
            bestv_r = jnp.where(upd, iou, bestv_r)
        key3_r = (img_r.astype(jnp.int32) * (_NA * _S)
                  + best_r * _S + cell_rf.astype(jnp.int32))
        dup = ((key3_c == key3_r) & (jt > iota_c)).astype(jnp.float32)
        later = jnp.max(dup, axis=1, keepdims=True)
        valid = jnp.where(later > 0.0, 0.0, 1.0)

        tx = gx_c - cellx_c
        ty = gy_c - celly_c
        tw = gw_c / baw_c
        th = gh_c / bah_c
        pw = w_p * (1.0 / _NX)
        ph = h_p * (1.0 / _NY)
        bscale = 2.0 - pw * ph
        box = bscale * ((sx_g - tx) ** 2 + (sy_g - ty) ** 2
                        + (ew_g - tw) ** 2 + (eh_g - th) ** 2)
        obj = _OBJ_SCALE * (conf_g - miou_g) ** 2

        m = jnp.max(logits, axis=1, keepdims=True)
        lse = m + jnp.log(jnp.sum(jnp.exp(logits - m), axis=1, keepdims=True))
        ci = jax.lax.broadcasted_iota(jnp.int32, (_T, _NC), 1)
        sel = jnp.sum(jnp.where(ci == cls_c, logits, 0.0), axis=1, keepdims=True)
        ce = lse - sel

        nocorr = jnp.where(miou_g < _IGNORE, conf_g * conf_g, 0.0)
        possum = jnp.sum(valid * (box + obj + ce - nocorr),
                         axis=(0, 1), keepdims=True)
        nsum = jnp.sum(nacc_ref[...], axis=(0, 1), keepdims=True) * (1.0 / 8.0)
        out_ref[...] = (nsum + possum) / _BS


def kernel(p, targets, anchors):
    x = p.reshape(_BS, _CH, _S)
    tcol = targets
    trow = targets.T
    out = pl.pallas_call(
        _loss_kernel,
        grid=(_BS // _IPS,),
        in_specs=[
            pl.BlockSpec((_IPS, _CH, _S), lambda b: (b, 0, 0)),
            pl.BlockSpec((_T, 6), lambda b: (0, 0)),
            pl.BlockSpec((6, _T), lambda b: (0, 0)),
            pl.BlockSpec((_NA, 2), lambda b: (0, 0)),
        ],
        out_specs=pl.BlockSpec((1, 1), lambda b: (0, 0)),
        out_shape=jax.ShapeDtypeStruct((1, 1), jnp.float32),
        scratch_shapes=[
            pltpu.VMEM((_T, 8), jnp.float32),
            pltpu.VMEM((_T, _CH), jnp.float32),
            pltpu.VMEM((8, _S), jnp.float32),
        ],
        compiler_params=pltpu.CompilerParams(
            dimension_semantics=("arbitrary",)),
        interpret=_INTERPRET,
    )(x, tcol, trow, anchors)
    return out[0, 0]
